# Initial kernel scaffold; baseline (speedup 1.0000x reference)
import jax
import jax.numpy as jnp
from jax.experimental import pallas as pl


def kernel(usr_id, item_ids, adj_ent, adj_rel, usr_table, ent_table, rel_table, W, b):
    raise NotImplementedError("write your pallas kernel here")


# speedup vs baseline: 4.6643x; 4.6643x over previous
import functools

import jax
import jax.numpy as jnp
from jax import lax
from jax.experimental import pallas as pl
from jax.experimental.pallas import tpu as pltpu
from jax.experimental.pallas import tpu_sc as plsc

NUM_REL = 32
DIM = 64
NN = 16


def _sc_gather(item_ids, usr_id, adj_ent, adj_rel, ent_table, usr_table):
    B = item_ids.shape[0]
    info = plsc.get_sparse_core_info()
    NC, NS = info.num_cores, info.num_subcores
    NW = NC * NS
    bpw = B // NW

    mesh = plsc.VectorSubcoreMesh(core_axis_name="c", subcore_axis_name="s")

    out_type = (
        jax.ShapeDtypeStruct((B, DIM), jnp.float32),
        jax.ShapeDtypeStruct((B, DIM), jnp.float32),
        jax.ShapeDtypeStruct((B, NN, DIM), jnp.float32),
        jax.ShapeDtypeStruct((B, NN * NN, DIM), jnp.float32),
        jax.ShapeDtypeStruct((B, NN), jnp.int32),
        jax.ShapeDtypeStruct((B, NN, NN), jnp.int32),
    )
    scratch = [
        pltpu.VMEM((bpw,), jnp.int32),
        pltpu.VMEM((bpw,), jnp.int32),
        pltpu.VMEM((bpw, NN), jnp.int32),
        pltpu.VMEM((bpw, NN), jnp.int32),
        pltpu.VMEM((bpw, DIM), jnp.float32),
        pltpu.VMEM((bpw, DIM), jnp.float32),
        pltpu.VMEM((NN, NN), jnp.int32),
        pltpu.VMEM((NN, NN), jnp.int32),
        pltpu.VMEM((2, 128), jnp.int32),
        pltpu.VMEM((NN, DIM), jnp.float32),
        pltpu.VMEM((NN * NN, DIM), jnp.float32),
        pltpu.SemaphoreType.DMA,
        pltpu.SemaphoreType.DMA,
        pltpu.SemaphoreType.DMA,
    ]

    @functools.partial(pl.kernel, out_type=out_type, mesh=mesh,
                       scratch_types=scratch,
                       compiler_params=pltpu.CompilerParams(
                           use_tc_tiling_on_sc=False))
    def k(item_h, usr_h, adj_ent_h, adj_rel_h, ent_h, usrt_h,
          u_o, e0_o, e1_o, e2_o, r1_o, r2_o,
          it_v, us_v, a1_v, r1_v, e0_v, u_v, a2_v, r2_v, flat_v, e1_v, e2_v,
          sem, sem_b, sem_c):
        wid = lax.axis_index("s") * NC + lax.axis_index("c")
        base = wid * bpw

        pltpu.sync_copy(item_h.at[pl.ds(base, bpw)], it_v)
        pltpu.sync_copy(usr_h.at[pl.ds(base, bpw)], us_v)
        pltpu.async_copy(adj_ent_h.at[it_v], a1_v, sem).wait()
        pltpu.async_copy(adj_rel_h.at[it_v], r1_v, sem).wait()
        pltpu.async_copy(ent_h.at[it_v], e0_v, sem).wait()
        pltpu.async_copy(usrt_h.at[us_v], u_v, sem).wait()
        pltpu.sync_copy(r1_v, r1_o.at[pl.ds(base, bpw)])
        pltpu.sync_copy(e0_v, e0_o.at[pl.ds(base, bpw)])
        pltpu.sync_copy(u_v, u_o.at[pl.ds(base, bpw)])

        def row(i, carry):
            gb = base + i
            ids1 = a1_v.at[i]
            h_a2 = pltpu.async_copy(adj_ent_h.at[ids1], a2_v, sem)
            h_e1 = pltpu.async_copy(ent_h.at[ids1], e1_v, sem_b)
            h_r2 = pltpu.async_copy(adj_rel_h.at[ids1], r2_v, sem_b)
            h_a2.wait()
            for j in range(NN):
                flat_v[j // 8, pl.ds((j % 8) * NN, NN)] = a2_v[j, :]
            h_lo = pltpu.async_copy(ent_h.at[flat_v.at[0]],
                                    e2_v.at[pl.ds(0, 128)], sem_c)
            h_hi = pltpu.async_copy(ent_h.at[flat_v.at[1]],
                                    e2_v.at[pl.ds(128, 128)], sem_c)
            h_e1.wait()
            h_r2.wait()
            pltpu.sync_copy(e1_v, e1_o.at[gb])
            pltpu.sync_copy(r2_v, r2_o.at[gb])
            h_lo.wait()
            h_hi.wait()
            pltpu.sync_copy(e2_v, e2_o.at[gb])
            return carry

        lax.fori_loop(0, bpw, row, 0)

    return k(item_ids, usr_id, adj_ent, adj_rel, ent_table, usr_table)


def _tc_body(u_r, e0_r, e1_r, e2_r, r1_r, r2_r, rel_r, W_r, b_r, out_r):
    u = u_r[...]
    P = jnp.dot(u, rel_r[...].T, preferred_element_type=jnp.float32)
    r1 = r1_r[...]
    r2 = r2_r[...]

    Bb = u.shape[0]
    NG = NN * NN
    s1 = jnp.zeros((Bb, NN), jnp.float32)
    s2 = jnp.zeros((Bb, NG), jnp.float32)
    for kk in range(NUM_REL):
        pk = P[:, kk][:, None]
        s1 = s1 + jnp.where(r1 == kk, pk, 0.0)
        s2 = s2 + jnp.where(r2 == kk, pk, 0.0)

    E1 = jnp.exp(s1)
    w1 = E1 / jnp.sum(E1, axis=-1, keepdims=True)
    E2 = jnp.exp(s2)
    grp = jax.lax.broadcasted_iota(jnp.int32, (NG, NN), 0) // NN
    G = (grp == jax.lax.broadcasted_iota(jnp.int32, (NG, NN), 1)).astype(jnp.float32)
    GT = (jax.lax.broadcasted_iota(jnp.int32, (NN, NG), 0)
          == jax.lax.broadcasted_iota(jnp.int32, (NN, NG), 1) // NN).astype(jnp.float32)
    denom = jnp.dot(E2, G, preferred_element_type=jnp.float32)
    denb = jnp.dot(denom, GT, preferred_element_type=jnp.float32)
    w2 = E2 / denb

    e1 = e1_r[...]
    e2 = e2_r[...]

    A = GT[None, :, :] * w2[:, None, :]
    nagg = jax.lax.dot_general(
        A, e2, (((2,), (1,)), ((0,), (0,))),
        preferred_element_type=jnp.float32)
    agg1 = e1 + nagg

    agg0 = e0_r[...]
    for n in range(NN):
        agg0 = agg0 + w1[:, n][:, None] * e1[:, n, :]

    Wt = W_r[...].T
    bb = b_r[...]
    h0 = jax.nn.sigmoid(jnp.dot(agg0, Wt, preferred_element_type=jnp.float32) + bb)
    h1 = jax.nn.sigmoid(
        (jnp.dot(agg1.reshape(Bb * NN, DIM), Wt,
                 preferred_element_type=jnp.float32) + bb).reshape(Bb, NN, DIM))

    acc0 = jnp.zeros((Bb, DIM), jnp.float32)
    for n in range(NN):
        acc0 = acc0 + w1[:, n][:, None] * h1[:, n, :]
    f = jnp.tanh(jnp.dot(h0 + acc0, Wt, preferred_element_type=jnp.float32) + bb)

    out_r[...] = jax.nn.sigmoid(jnp.sum(u * f, axis=-1))


def _tc_dense(u, e0, e1, e2, r1, r2, rel_table, W, b):
    B = u.shape[0]
    Bb = 128
    grid = (B // Bb,)
    return pl.pallas_call(
        _tc_body,
        grid=grid,
        compiler_params=pltpu.CompilerParams(
            vmem_limit_bytes=100 * 1024 * 1024),
        in_specs=[
            pl.BlockSpec((Bb, DIM), lambda i: (i, 0)),
            pl.BlockSpec((Bb, DIM), lambda i: (i, 0)),
            pl.BlockSpec((Bb, NN, DIM), lambda i: (i, 0, 0)),
            pl.BlockSpec((Bb, NN * NN, DIM), lambda i: (i, 0, 0)),
            pl.BlockSpec((Bb, NN), lambda i: (i, 0)),
            pl.BlockSpec((Bb, NN * NN), lambda i: (i, 0)),
            pl.BlockSpec((NUM_REL, DIM), lambda i: (0, 0)),
            pl.BlockSpec((DIM, DIM), lambda i: (0, 0)),
            pl.BlockSpec((1, DIM), lambda i: (0, 0)),
        ],
        out_specs=pl.BlockSpec((Bb,), lambda i: (i,)),
        out_shape=jax.ShapeDtypeStruct((B,), jnp.float32),
    )(u, e0, e1, e2, r1, r2, rel_table, W, b)


def kernel(usr_id, item_ids, adj_ent, adj_rel, usr_table, ent_table, rel_table, W, b):
    B = usr_id.shape[0]
    item_flat = item_ids.reshape(B).astype(jnp.int32)
    usr_flat = usr_id.reshape(B).astype(jnp.int32)
    adj_ent = adj_ent.astype(jnp.int32)
    adj_rel = adj_rel.astype(jnp.int32)

    u, e0, e1, e2, r1, r2 = _sc_gather(
        item_flat, usr_flat, adj_ent, adj_rel, ent_table, usr_table)

    return _tc_dense(u, e0, e1, e2, r1, r2.reshape(B, NN * NN), rel_table,
                     W, b.reshape(1, DIM))


# speedup vs baseline: 6.6274x; 1.4209x over previous
import functools

import jax
import jax.numpy as jnp
from jax import lax
from jax.experimental import pallas as pl
from jax.experimental.pallas import tpu as pltpu
from jax.experimental.pallas import tpu_sc as plsc

NUM_REL = 32
DIM = 64
NN = 16


def _sc_fused(item_ids, usr_id, adj_ent, adj_rel, ent_table, usr_table, rel_table):
    B = item_ids.shape[0]
    info = plsc.get_sparse_core_info()
    NC, NS = info.num_cores, info.num_subcores
    NW = NC * NS
    bpw = B // NW

    mesh = plsc.VectorSubcoreMesh(core_axis_name="c", subcore_axis_name="s")

    out_type = (
        jax.ShapeDtypeStruct((B, DIM), jnp.float32),
        jax.ShapeDtypeStruct((B, DIM), jnp.float32),
        jax.ShapeDtypeStruct((B, DIM), jnp.float32),
        jax.ShapeDtypeStruct((B, NN, DIM), jnp.float32),
        jax.ShapeDtypeStruct((B, NN), jnp.float32),
    )
    f32, i32 = jnp.float32, jnp.int32
    scratch = [
        pltpu.VMEM((bpw,), i32),
        pltpu.VMEM((bpw,), i32),
        pltpu.VMEM((bpw, NN), i32),
        pltpu.VMEM((bpw, NN), i32),
        pltpu.VMEM((bpw, DIM), f32),
        pltpu.VMEM((bpw, DIM), f32),
        pltpu.VMEM((NUM_REL, DIM), f32),
        pltpu.VMEM((NUM_REL, bpw), f32),
        pltpu.VMEM((NN,), f32),
        [pltpu.VMEM((NN, NN), i32)] * 2,
        [pltpu.VMEM((NN, NN), i32)] * 2,
        [pltpu.VMEM((2, 128), i32)] * 2,
        [pltpu.VMEM((NN, DIM), f32)] * 2,
        [pltpu.VMEM((NN * NN, DIM), f32)] * 2,
        [pltpu.VMEM((NN,), f32)] * 2,
        [pltpu.VMEM((DIM,), f32)] * 2,
        [pltpu.VMEM((NN, DIM), f32)] * 2,
        pltpu.SemaphoreType.DMA,
        [pltpu.SemaphoreType.DMA] * 2,
        [pltpu.SemaphoreType.DMA] * 2,
        [pltpu.SemaphoreType.DMA] * 2,
        [pltpu.SemaphoreType.DMA] * 2,
    ]

    @functools.partial(pl.kernel, out_type=out_type, mesh=mesh,
                       scratch_types=scratch,
                       compiler_params=pltpu.CompilerParams(
                           use_tc_tiling_on_sc=False,
                           needs_layout_passes=False))
    def k(item_h, usr_h, adj_ent_h, adj_rel_h, ent_h, usrt_h, rel_h,
          u_o, e0_o, s0_o, agg1_o, w1_o,
          it_v, us_v, a1_v, r1_v, e0_v, u_v, rel_v, p_v, ebuf,
          a2_s, r2_s, flat_s, e1_s, e2_s, w1buf_s, s0buf_s, aggbuf_s,
          sem_hdr, sem_a2, sem_er, sem_e2, sem_out):
        wid = lax.axis_index("s") * NC + lax.axis_index("c")
        base = wid * bpw
        iota16 = lax.iota(i32, NN)
        zeros16 = jnp.zeros((NN,), f32)

        pltpu.sync_copy(item_h.at[pl.ds(base, bpw)], it_v)
        pltpu.sync_copy(usr_h.at[pl.ds(base, bpw)], us_v)
        h1 = pltpu.async_copy(adj_ent_h.at[it_v], a1_v, sem_hdr)
        h2 = pltpu.async_copy(adj_rel_h.at[it_v], r1_v, sem_hdr)
        h3 = pltpu.async_copy(ent_h.at[it_v], e0_v, sem_hdr)
        h4 = pltpu.async_copy(usrt_h.at[us_v], u_v, sem_hdr)
        pltpu.sync_copy(rel_h, rel_v)
        h1.wait(); h2.wait(); h3.wait(); h4.wait()
        pltpu.sync_copy(e0_v, e0_o.at[pl.ds(base, bpw)])
        pltpu.sync_copy(u_v, u_o.at[pl.ds(base, bpw)])

        for kk in range(NUM_REL):
            p_v[kk, pl.ds(0, NN)] = zeros16
            p_v[kk, pl.ds(NN, NN)] = zeros16

        def pbody(d, carry):
            dvec = jnp.full((NN,), d, i32)
            ucol0 = plsc.load_gather(u_v, [iota16, dvec])
            ucol1 = plsc.load_gather(u_v, [iota16 + NN, dvec])
            for kk in range(NUM_REL):
                wvec = plsc.load_gather(rel_v, [jnp.full((NN,), kk, i32), dvec])
                plsc.addupdate(p_v.at[kk, pl.ds(0, NN)], ucol0 * wvec)
                plsc.addupdate(p_v.at[kk, pl.ds(NN, NN)], ucol1 * wvec)
            return carry

        lax.fori_loop(0, DIM, pbody, 0)

        def fire_a2(i, s):
            return pltpu.async_copy(adj_ent_h.at[a1_v.at[i]], a2_s[s], sem_a2[s])

        def fire_er(i, s):
            pltpu.async_copy(ent_h.at[a1_v.at[i]], e1_s[s], sem_er[s])
            pltpu.async_copy(adj_rel_h.at[a1_v.at[i]], r2_s[s], sem_er[s])

        def wait_a2(i, s):
            pltpu.make_async_copy(adj_ent_h.at[a1_v.at[i]], a2_s[s], sem_a2[s]).wait()

        def flatten_fire_e2(i, s):
            for j in range(NN):
                flat_s[s][j // 8, pl.ds((j % 8) * NN, NN)] = a2_s[s][j, :]
            pltpu.async_copy(ent_h.at[flat_s[s].at[0]],
                             e2_s[s].at[pl.ds(0, 128)], sem_e2[s])
            pltpu.async_copy(ent_h.at[flat_s[s].at[1]],
                             e2_s[s].at[pl.ds(128, 128)], sem_e2[s])

        def drain_outs(i, s):
            gbp = base + i
            pltpu.make_async_copy(w1buf_s[s], w1_o.at[gbp], sem_out[s]).wait()
            pltpu.make_async_copy(s0buf_s[s], s0_o.at[gbp], sem_out[s]).wait()
            pltpu.make_async_copy(aggbuf_s[s], agg1_o.at[gbp], sem_out[s]).wait()

        def compute(i, s):
            gb = base + i

            @pl.when(i >= 2)
            def _():
                drain_outs(i - 2, s)

            pltpu.make_async_copy(ent_h.at[a1_v.at[i]], e1_s[s], sem_er[s]).wait()
            pltpu.make_async_copy(adj_rel_h.at[a1_v.at[i]], r2_s[s], sem_er[s]).wait()
            pltpu.make_async_copy(ent_h.at[flat_s[s].at[0]],
                                  e2_s[s].at[pl.ds(0, 128)], sem_e2[s]).wait()
            pltpu.make_async_copy(ent_h.at[flat_s[s].at[1]],
                                  e2_s[s].at[pl.ds(128, 128)], sem_e2[s]).wait()

            ivec = jnp.full((NN,), i, i32)
            r1vec = plsc.load_gather(r1_v, [ivec, iota16])
            sc1 = plsc.load_gather(p_v, [r1vec, ivec])
            es1 = jnp.exp(sc1)
            w1vec = es1 / jnp.sum(es1)
            w1buf_s[s][...] = w1vec
            acc0 = [zeros16] * 4
            for n in range(NN):
                w = plsc.load_gather(w1buf_s[s], [jnp.full((NN,), n, i32)])
                for c in range(4):
                    acc0[c] = acc0[c] + w * e1_s[s][n, pl.ds(16 * c, 16)]
            for c in range(4):
                s0buf_s[s][pl.ds(16 * c, 16)] = acc0[c]

            for m in range(NN):
                r2vec = r2_s[s][m, :]
                sc2 = plsc.load_gather(p_v, [r2vec, ivec])
                es2 = jnp.exp(sc2)
                ssum2 = jnp.sum(es2)
                ebuf[...] = es2
                acc = [zeros16] * 4
                for n in range(NN):
                    w = plsc.load_gather(ebuf, [jnp.full((NN,), n, i32)])
                    row = m * NN + n
                    for c in range(4):
                        acc[c] = acc[c] + w * e2_s[s][row, pl.ds(16 * c, 16)]
                for c in range(4):
                    aggbuf_s[s][m, pl.ds(16 * c, 16)] = (
                        e1_s[s][m, pl.ds(16 * c, 16)] + acc[c] / ssum2)

            pltpu.async_copy(w1buf_s[s], w1_o.at[gb], sem_out[s])
            pltpu.async_copy(s0buf_s[s], s0_o.at[gb], sem_out[s])
            pltpu.async_copy(aggbuf_s[s], agg1_o.at[gb], sem_out[s])

        fire_a2(0, 0)
        fire_er(0, 0)
        wait_a2(0, 0)
        flatten_fire_e2(0, 0)
        fire_a2(1, 1)
        fire_er(1, 1)

        def body(j, carry):
            r0 = 2 * j
            more = j < (bpw // 2 - 1)

            @pl.when(more)
            def _():
                fire_a2(r0 + 2, 0)

            wait_a2(r0 + 1, 1)
            flatten_fire_e2(r0 + 1, 1)
            compute(r0, 0)

            @pl.when(more)
            def _():
                fire_er(r0 + 2, 0)
                wait_a2(r0 + 2, 0)
                flatten_fire_e2(r0 + 2, 0)
                fire_a2(r0 + 3, 1)

            compute(r0 + 1, 1)

            @pl.when(more)
            def _():
                fire_er(r0 + 3, 1)

            return carry

        lax.fori_loop(0, bpw // 2, body, 0)

        drain_outs(bpw - 2, 0)
        drain_outs(bpw - 1, 1)

    return k(item_ids, usr_id, adj_ent, adj_rel, ent_table, usr_table, rel_table)


def _tc_body(u_r, e0_r, s0_r, agg_r, w1_r, W_r, b_r, out_r):
    B = u_r.shape[0]
    Wt = W_r[...].T
    bb = b_r[...]
    h0 = jax.nn.sigmoid(
        jnp.dot(e0_r[...] + s0_r[...], Wt, preferred_element_type=jnp.float32) + bb)
    agg = agg_r[...]
    h1 = jax.nn.sigmoid(
        (jnp.dot(agg.reshape(B * NN, DIM), Wt,
                 preferred_element_type=jnp.float32) + bb).reshape(B, NN, DIM))
    w1 = w1_r[...]
    acc = jnp.zeros((B, DIM), jnp.float32)
    for n in range(NN):
        acc = acc + w1[:, n][:, None] * h1[:, n, :]
    f = jnp.tanh(jnp.dot(h0 + acc, Wt, preferred_element_type=jnp.float32) + bb)
    out_r[...] = jax.nn.sigmoid(jnp.sum(u_r[...] * f, axis=-1))


def _tc_dense(u, e0, s0, agg1, w1, W, b):
    B = u.shape[0]
    return pl.pallas_call(
        _tc_body,
        out_shape=jax.ShapeDtypeStruct((B,), jnp.float32),
        compiler_params=pltpu.CompilerParams(
            vmem_limit_bytes=100 * 1024 * 1024),
    )(u, e0, s0, agg1, w1, W, b)


def kernel(usr_id, item_ids, adj_ent, adj_rel, usr_table, ent_table, rel_table, W, b):
    B = usr_id.shape[0]
    item_flat = item_ids.reshape(B).astype(jnp.int32)
    usr_flat = usr_id.reshape(B).astype(jnp.int32)
    adj_ent = adj_ent.astype(jnp.int32)
    adj_rel = adj_rel.astype(jnp.int32)

    u, e0, s0, agg1, w1 = _sc_fused(
        item_flat, usr_flat, adj_ent, adj_rel, ent_table, usr_table, rel_table)

    return _tc_dense(u, e0, s0, agg1, w1, W, b.reshape(1, DIM))
